# Initial kernel scaffold; baseline (speedup 1.0000x reference)
import jax
import jax.numpy as jnp
from jax.experimental import pallas as pl


def kernel(x, x_res, batch, batch_size, W_pre, b_pre, W_gate, W_value, W_post, b_post, scale):
    raise NotImplementedError("write your pallas kernel here")


# speedup vs baseline: 1.7624x; 1.7624x over previous
import functools

import jax
import jax.numpy as jnp
from jax import lax
from jax.experimental import pallas as pl
from jax.experimental.pallas import tpu as pltpu
from jax.experimental.pallas import tpu_sc as plsc

N, B, WIDTH = 100000, 1024, 128
G, HIN, WS = 8, 16, 2
HOUT = HIN * WS

NC, NS = 2, 16
NW = NC * NS
CH = 128
NFULL = N // CH
TAIL = N - NFULL * CH
KMAX = (NFULL + NW - 1) // NW
ROWS_PER_SUB = B // NS


def _sc_mesh():
    return plsc.VectorSubcoreMesh(core_axis_name="c", subcore_axis_name="s")


def _sc_scatter(x, batch):
    @functools.partial(
        pl.kernel,
        out_type=jax.ShapeDtypeStruct((NC, B, WIDTH), jnp.float32),
        mesh=_sc_mesh(),
        scratch_types=[
            pltpu.VMEM((CH,), jnp.int32),
            pltpu.VMEM((CH, WIDTH), jnp.float32),
            pltpu.VMEM((TAIL,), jnp.int32),
            pltpu.VMEM((TAIL, WIDTH), jnp.float32),
            pltpu.VMEM((8, WIDTH), jnp.float32),
            pltpu.VMEM_SHARED((B, WIDTH), jnp.float32),
        ],
    )
    def k(x_hbm, batch_hbm, part_hbm, idx_v, rows_v, idx_t, rows_t, zbuf, acc_sh):
        cid = lax.axis_index("c")
        sid = lax.axis_index("s")
        wid = sid * NC + cid

        z16 = jnp.zeros((16,), jnp.float32)
        for r in range(8):
            for c8 in range(WIDTH // 16):
                zbuf[r, pl.ds(c8 * 16, 16)] = z16
        for t in range(ROWS_PER_SUB // 8):
            pltpu.sync_copy(zbuf, acc_sh.at[pl.ds(sid * ROWS_PER_SUB + t * 8, 8)])
        plsc.subcore_barrier()

        def chunk_body(kk, carry):
            c = wid + kk * NW

            @pl.when(c < NFULL)
            def _():
                base = pl.multiple_of(c * CH, CH)
                pltpu.sync_copy(batch_hbm.at[pl.ds(base, CH)], idx_v)
                pltpu.sync_copy(x_hbm.at[pl.ds(base, CH)], rows_v)
                pltpu.sync_copy(rows_v, acc_sh.at[idx_v], add=True)

            return carry

        lax.fori_loop(0, KMAX, chunk_body, 0)

        @pl.when(wid == NW - 1)
        def _():
            pltpu.sync_copy(batch_hbm.at[pl.ds(NFULL * CH, TAIL)], idx_t)
            pltpu.sync_copy(x_hbm.at[pl.ds(NFULL * CH, TAIL)], rows_t)
            pltpu.sync_copy(rows_t, acc_sh.at[idx_t], add=True)

        plsc.subcore_barrier()
        pltpu.sync_copy(
            acc_sh.at[pl.ds(sid * ROWS_PER_SUB, ROWS_PER_SUB)],
            part_hbm.at[cid, pl.ds(sid * ROWS_PER_SUB, ROWS_PER_SUB)],
        )

    return k(x, batch)


def _sc_gather(xx, batch):
    @functools.partial(
        pl.kernel,
        out_type=jax.ShapeDtypeStruct((N, WIDTH), jnp.float32),
        mesh=_sc_mesh(),
        scratch_types=[
            pltpu.VMEM((CH,), jnp.int32),
            pltpu.VMEM((CH, WIDTH), jnp.float32),
            pltpu.VMEM((TAIL,), jnp.int32),
            pltpu.VMEM((TAIL, WIDTH), jnp.float32),
            pltpu.SemaphoreType.DMA,
        ],
    )
    def k(xx_hbm, batch_hbm, out_hbm, idx_v, rows_v, idx_t, rows_t, sem):
        cid = lax.axis_index("c")
        sid = lax.axis_index("s")
        wid = sid * NC + cid

        def chunk_body(kk, carry):
            c = wid + kk * NW

            @pl.when(c < NFULL)
            def _():
                base = pl.multiple_of(c * CH, CH)
                pltpu.sync_copy(batch_hbm.at[pl.ds(base, CH)], idx_v)
                pltpu.async_copy(xx_hbm.at[idx_v], rows_v, sem).wait()
                pltpu.sync_copy(rows_v, out_hbm.at[pl.ds(base, CH)])

            return carry

        lax.fori_loop(0, KMAX, chunk_body, 0)

        @pl.when(wid == NW - 1)
        def _():
            pltpu.sync_copy(batch_hbm.at[pl.ds(NFULL * CH, TAIL)], idx_t)
            pltpu.async_copy(xx_hbm.at[idx_t], rows_t, sem).wait()
            pltpu.sync_copy(rows_t, out_hbm.at[pl.ds(NFULL * CH, TAIL)])

    return k(xx, batch)


def _mlp_body(part, x_res, wpre_t, b_pre, proj, wg, wv, wpost_t, b_post, scale,
              xrn_out, xx_out):
    xr = part[0] + part[1] + x_res[...]
    xrn_out[...] = xr
    h = jnp.dot(xr, wpre_t[...], preferred_element_type=jnp.float32) + b_pre[...]
    mb = jnp.dot(h, proj[...], preferred_element_type=jnp.float32)
    d = h - mb
    var = jnp.dot(d * d, proj[...], preferred_element_type=jnp.float32)
    hn = d * lax.rsqrt(var + 1e-5)
    gate = jnp.dot(hn, wg[...], preferred_element_type=jnp.float32)
    val = jnp.dot(hn, wv[...], preferred_element_type=jnp.float32)
    z = jnp.maximum(gate, 0.0) * val
    y = jnp.dot(z, wpost_t[...], preferred_element_type=jnp.float32) + b_post[...]
    xx_out[...] = y * jnp.exp(scale[...])


def _tc_mlp(part, x_res, W_pre, b_pre, W_gate, W_value, W_post, b_post, scale):
    proj = jnp.kron(jnp.eye(G, dtype=jnp.float32),
                    jnp.ones((HIN, HIN), jnp.float32) / HIN)
    eye_g = jnp.eye(G, dtype=jnp.float32)
    wg = jnp.einsum('goi,gh->giho', W_gate, eye_g).reshape(G * HIN, G * HOUT)
    wv = jnp.einsum('goi,gh->giho', W_value, eye_g).reshape(G * HIN, G * HOUT)
    return pl.pallas_call(
        _mlp_body,
        out_shape=[
            jax.ShapeDtypeStruct((B, WIDTH), jnp.float32),
            jax.ShapeDtypeStruct((B, WIDTH), jnp.float32),
        ],
    )(part, x_res, W_pre.T, b_pre.reshape(1, WIDTH), proj, wg, wv,
      W_post.T, b_post.reshape(1, WIDTH), scale.reshape(1, WIDTH))


def kernel(x, x_res, batch, batch_size, W_pre, b_pre, W_gate, W_value, W_post,
           b_post, scale):
    del batch_size
    batch = batch.astype(jnp.int32)
    part = _sc_scatter(x, batch)
    xrn, xx = _tc_mlp(part, x_res, W_pre, b_pre, W_gate, W_value, W_post,
                      b_post, scale)
    out = _sc_gather(xx, batch)
    return (out, xrn)


# speedup vs baseline: 4.0155x; 2.2784x over previous
import functools

import jax
import jax.numpy as jnp
from jax import lax
from jax.experimental import pallas as pl
from jax.experimental.pallas import tpu as pltpu
from jax.experimental.pallas import tpu_sc as plsc

N, B, WIDTH = 100000, 1024, 128
G, HIN, WS = 8, 16, 2
HOUT = HIN * WS

NC, NS = 2, 16
NW = NC * NS
CH = 128
NFULL = N // CH
TAIL = N - NFULL * CH
KMAX = (NFULL + NW - 1) // NW
ROWS_PER_SUB = B // NS


def _sc_mesh():
    return plsc.VectorSubcoreMesh(core_axis_name="c", subcore_axis_name="s")


def _sc_scatter(x, batch):
    @functools.partial(
        pl.kernel,
        out_type=jax.ShapeDtypeStruct((NC, B, WIDTH), jnp.float32),
        mesh=_sc_mesh(),
        scratch_types=[
            pltpu.VMEM((CH,), jnp.int32),
            pltpu.VMEM((CH, WIDTH), jnp.float32),
            pltpu.VMEM((TAIL,), jnp.int32),
            pltpu.VMEM((TAIL, WIDTH), jnp.float32),
            pltpu.VMEM((8, WIDTH), jnp.float32),
            pltpu.VMEM_SHARED((B, WIDTH), jnp.float32),
        ],
    )
    def k(x_hbm, batch_hbm, part_hbm, idx_v, rows_v, idx_t, rows_t, zbuf, acc_sh):
        cid = lax.axis_index("c")
        sid = lax.axis_index("s")
        wid = sid * NC + cid

        z16 = jnp.zeros((16,), jnp.float32)
        for r in range(8):
            for c8 in range(WIDTH // 16):
                zbuf[r, pl.ds(c8 * 16, 16)] = z16
        for t in range(ROWS_PER_SUB // 8):
            pltpu.sync_copy(zbuf, acc_sh.at[pl.ds(sid * ROWS_PER_SUB + t * 8, 8)])
        plsc.subcore_barrier()

        def chunk_body(kk, carry):
            c = wid + kk * NW

            @pl.when(c < NFULL)
            def _():
                base = pl.multiple_of(c * CH, CH)
                pltpu.sync_copy(batch_hbm.at[pl.ds(base, CH)], idx_v)
                pltpu.sync_copy(x_hbm.at[pl.ds(base, CH)], rows_v)
                pltpu.sync_copy(rows_v, acc_sh.at[idx_v], add=True)

            return carry

        lax.fori_loop(0, KMAX, chunk_body, 0)

        @pl.when(wid == NW - 1)
        def _():
            pltpu.sync_copy(batch_hbm.at[pl.ds(NFULL * CH, TAIL)], idx_t)
            pltpu.sync_copy(x_hbm.at[pl.ds(NFULL * CH, TAIL)], rows_t)
            pltpu.sync_copy(rows_t, acc_sh.at[idx_t], add=True)

        plsc.subcore_barrier()
        pltpu.sync_copy(
            acc_sh.at[pl.ds(sid * ROWS_PER_SUB, ROWS_PER_SUB)],
            part_hbm.at[cid, pl.ds(sid * ROWS_PER_SUB, ROWS_PER_SUB)],
        )

    return k(x, batch)


def _sc_gather(xx, batch):
    @functools.partial(
        pl.kernel,
        out_type=jax.ShapeDtypeStruct((N, WIDTH), jnp.float32),
        mesh=_sc_mesh(),
        scratch_types=[
            pltpu.VMEM((CH,), jnp.int32),
            pltpu.VMEM((CH, WIDTH), jnp.float32),
            pltpu.VMEM((TAIL,), jnp.int32),
            pltpu.VMEM((TAIL, WIDTH), jnp.float32),
            pltpu.VMEM_SHARED((B, WIDTH), jnp.float32),
            pltpu.SemaphoreType.DMA,
        ],
    )
    def k(xx_hbm, batch_hbm, out_hbm, idx_v, rows_v, idx_t, rows_t, xx_sh, sem):
        cid = lax.axis_index("c")
        sid = lax.axis_index("s")
        wid = sid * NC + cid

        pltpu.sync_copy(
            xx_hbm.at[pl.ds(sid * ROWS_PER_SUB, ROWS_PER_SUB)],
            xx_sh.at[pl.ds(sid * ROWS_PER_SUB, ROWS_PER_SUB)],
        )
        plsc.subcore_barrier()

        def chunk_body(kk, carry):
            c = wid + kk * NW

            @pl.when(c < NFULL)
            def _():
                base = pl.multiple_of(c * CH, CH)
                pltpu.sync_copy(batch_hbm.at[pl.ds(base, CH)], idx_v)
                pltpu.async_copy(xx_sh.at[idx_v], rows_v, sem).wait()
                pltpu.sync_copy(rows_v, out_hbm.at[pl.ds(base, CH)])

            return carry

        lax.fori_loop(0, KMAX, chunk_body, 0)

        @pl.when(wid == NW - 1)
        def _():
            pltpu.sync_copy(batch_hbm.at[pl.ds(NFULL * CH, TAIL)], idx_t)
            pltpu.async_copy(xx_sh.at[idx_t], rows_t, sem).wait()
            pltpu.sync_copy(rows_t, out_hbm.at[pl.ds(NFULL * CH, TAIL)])

    return k(xx, batch)


def _mlp_body(part, x_res, wpre_t, b_pre, proj, wg, wv, wpost_t, b_post, scale,
              xrn_out, xx_out):
    xr = part[0] + part[1] + x_res[...]
    xrn_out[...] = xr
    h = jnp.dot(xr, wpre_t[...], preferred_element_type=jnp.float32) + b_pre[...]
    mb = jnp.dot(h, proj[...], preferred_element_type=jnp.float32)
    d = h - mb
    var = jnp.dot(d * d, proj[...], preferred_element_type=jnp.float32)
    hn = d * lax.rsqrt(var + 1e-5)
    gate = jnp.dot(hn, wg[...], preferred_element_type=jnp.float32)
    val = jnp.dot(hn, wv[...], preferred_element_type=jnp.float32)
    z = jnp.maximum(gate, 0.0) * val
    y = jnp.dot(z, wpost_t[...], preferred_element_type=jnp.float32) + b_post[...]
    xx_out[...] = y * jnp.exp(scale[...])


def _tc_mlp(part, x_res, W_pre, b_pre, W_gate, W_value, W_post, b_post, scale):
    proj = jnp.kron(jnp.eye(G, dtype=jnp.float32),
                    jnp.ones((HIN, HIN), jnp.float32) / HIN)
    eye_g = jnp.eye(G, dtype=jnp.float32)
    wg = jnp.einsum('goi,gh->giho', W_gate, eye_g).reshape(G * HIN, G * HOUT)
    wv = jnp.einsum('goi,gh->giho', W_value, eye_g).reshape(G * HIN, G * HOUT)
    return pl.pallas_call(
        _mlp_body,
        out_shape=[
            jax.ShapeDtypeStruct((B, WIDTH), jnp.float32),
            jax.ShapeDtypeStruct((B, WIDTH), jnp.float32),
        ],
    )(part, x_res, W_pre.T, b_pre.reshape(1, WIDTH), proj, wg, wv,
      W_post.T, b_post.reshape(1, WIDTH), scale.reshape(1, WIDTH))


def kernel(x, x_res, batch, batch_size, W_pre, b_pre, W_gate, W_value, W_post,
           b_post, scale):
    del batch_size
    batch = batch.astype(jnp.int32)
    part = _sc_scatter(x, batch)
    xrn, xx = _tc_mlp(part, x_res, W_pre, b_pre, W_gate, W_value, W_post,
                      b_post, scale)
    out = _sc_gather(xx, batch)
    return (out, xrn)


# speedup vs baseline: 6.7876x; 1.6904x over previous
import functools

import jax
import jax.numpy as jnp
from jax import lax
from jax.experimental import pallas as pl
from jax.experimental.pallas import tpu as pltpu
from jax.experimental.pallas import tpu_sc as plsc

N, B, WIDTH = 100000, 1024, 128
G, HIN, WS = 8, 16, 2
HOUT = HIN * WS

NC, NS = 2, 16
NW = NC * NS
CH = 128
NFULL = N // CH
TAIL = N - NFULL * CH
KMAX = (NFULL + NW - 1) // NW
ROWS_PER_SUB = B // NS


def _sc_mesh():
    return plsc.VectorSubcoreMesh(core_axis_name="c", subcore_axis_name="s")


def _sc_scatter(x, batch):
    @functools.partial(
        pl.kernel,
        out_type=jax.ShapeDtypeStruct((NC, B, WIDTH), jnp.float32),
        mesh=_sc_mesh(),
        scratch_types=[
            pltpu.VMEM((2, CH), jnp.int32),
            pltpu.VMEM((2, CH, WIDTH), jnp.float32),
            pltpu.VMEM((TAIL,), jnp.int32),
            pltpu.VMEM((TAIL, WIDTH), jnp.float32),
            pltpu.VMEM((8, WIDTH), jnp.float32),
            pltpu.VMEM_SHARED((B, WIDTH), jnp.float32),
            pltpu.SemaphoreType.DMA,
            pltpu.SemaphoreType.DMA,
        ],
    )
    def k(x_hbm, batch_hbm, part_hbm, idx2, rows2, idx_t, rows_t, zbuf, acc_sh,
          isem, rsem):
        cid = lax.axis_index("c")
        sid = lax.axis_index("s")
        wid = sid * NC + cid

        base0 = pl.multiple_of(wid * CH, CH)
        pltpu.async_copy(batch_hbm.at[pl.ds(base0, CH)], idx2.at[0], isem)
        pltpu.async_copy(x_hbm.at[pl.ds(base0, CH)], rows2.at[0], rsem)

        z16 = jnp.zeros((16,), jnp.float32)
        for r in range(8):
            for c8 in range(WIDTH // 16):
                zbuf[r, pl.ds(c8 * 16, 16)] = z16
        for t in range(ROWS_PER_SUB // 8):
            pltpu.sync_copy(zbuf, acc_sh.at[pl.ds(sid * ROWS_PER_SUB + t * 8, 8)])
        plsc.subcore_barrier()

        def chunk_body(kk, carry):
            c = wid + kk * NW
            cn = c + NW
            b0 = lax.rem(kk, 2)
            b1 = lax.rem(kk + 1, 2)

            @pl.when(cn < NFULL)
            def _():
                basen = pl.multiple_of(cn * CH, CH)
                pltpu.async_copy(batch_hbm.at[pl.ds(basen, CH)], idx2.at[b1], isem)
                pltpu.async_copy(x_hbm.at[pl.ds(basen, CH)], rows2.at[b1], rsem)

            @pl.when(c < NFULL)
            def _():
                base = pl.multiple_of(c * CH, CH)
                pltpu.make_async_copy(batch_hbm.at[pl.ds(base, CH)], idx2.at[b0],
                                      isem).wait()
                pltpu.make_async_copy(x_hbm.at[pl.ds(base, CH)], rows2.at[b0],
                                      rsem).wait()
                pltpu.sync_copy(rows2.at[b0], acc_sh.at[idx2.at[b0]], add=True)

            return carry

        lax.fori_loop(0, KMAX, chunk_body, 0)

        @pl.when(wid == NW - 1)
        def _():
            pltpu.sync_copy(batch_hbm.at[pl.ds(NFULL * CH, TAIL)], idx_t)
            pltpu.sync_copy(x_hbm.at[pl.ds(NFULL * CH, TAIL)], rows_t)
            pltpu.sync_copy(rows_t, acc_sh.at[idx_t], add=True)

        plsc.subcore_barrier()
        pltpu.sync_copy(
            acc_sh.at[pl.ds(sid * ROWS_PER_SUB, ROWS_PER_SUB)],
            part_hbm.at[cid, pl.ds(sid * ROWS_PER_SUB, ROWS_PER_SUB)],
        )

    return k(x, batch)


def _sc_gather(xx, batch):
    @functools.partial(
        pl.kernel,
        out_type=jax.ShapeDtypeStruct((N, WIDTH), jnp.float32),
        mesh=_sc_mesh(),
        scratch_types=[
            pltpu.VMEM((2, CH), jnp.int32),
            pltpu.VMEM((2, CH, WIDTH), jnp.float32),
            pltpu.VMEM((TAIL,), jnp.int32),
            pltpu.VMEM((TAIL, WIDTH), jnp.float32),
            pltpu.VMEM_SHARED((B, WIDTH), jnp.float32),
            pltpu.SemaphoreType.DMA,
            pltpu.SemaphoreType.DMA,
            pltpu.SemaphoreType.DMA,
        ],
    )
    def k(xx_hbm, batch_hbm, out_hbm, idx2, rows2, idx_t, rows_t, xx_sh,
          isem, gsem, wsem):
        cid = lax.axis_index("c")
        sid = lax.axis_index("s")
        wid = sid * NC + cid

        base0 = pl.multiple_of(wid * CH, CH)
        pltpu.async_copy(batch_hbm.at[pl.ds(base0, CH)], idx2.at[0], isem)

        pltpu.sync_copy(
            xx_hbm.at[pl.ds(sid * ROWS_PER_SUB, ROWS_PER_SUB)],
            xx_sh.at[pl.ds(sid * ROWS_PER_SUB, ROWS_PER_SUB)],
        )
        plsc.subcore_barrier()

        def chunk_body(kk, carry):
            c = wid + kk * NW
            cn = c + NW
            b0 = lax.rem(kk, 2)
            b1 = lax.rem(kk + 1, 2)

            @pl.when(cn < NFULL)
            def _():
                basen = pl.multiple_of(cn * CH, CH)
                pltpu.async_copy(batch_hbm.at[pl.ds(basen, CH)], idx2.at[b1], isem)

            @pl.when(c < NFULL)
            def _():
                base = pl.multiple_of(c * CH, CH)
                @pl.when(kk >= 2)
                def _():
                    pltpu.make_async_copy(rows2.at[b0],
                                          out_hbm.at[pl.ds(base, CH)],
                                          wsem).wait()

                pltpu.make_async_copy(batch_hbm.at[pl.ds(base, CH)], idx2.at[b0],
                                      isem).wait()
                pltpu.async_copy(xx_sh.at[idx2.at[b0]], rows2.at[b0], gsem).wait()
                pltpu.async_copy(rows2.at[b0], out_hbm.at[pl.ds(base, CH)], wsem)

            return carry

        lax.fori_loop(0, KMAX, chunk_body, 0)

        for _ in range(2):
            pltpu.make_async_copy(rows2.at[0], out_hbm.at[pl.ds(0, CH)],
                                  wsem).wait()

        @pl.when(wid == NW - 1)
        def _():
            pltpu.sync_copy(batch_hbm.at[pl.ds(NFULL * CH, TAIL)], idx_t)
            pltpu.async_copy(xx_sh.at[idx_t], rows_t, gsem).wait()
            pltpu.sync_copy(rows_t, out_hbm.at[pl.ds(NFULL * CH, TAIL)])

    return k(xx, batch)


def _mlp_body(part, x_res, wpre_t, b_pre, proj, wg, wv, wpost_t, b_post, scale,
              xrn_out, xx_out):
    xr = part[0] + part[1] + x_res[...]
    xrn_out[...] = xr
    h = jnp.dot(xr, wpre_t[...], preferred_element_type=jnp.float32) + b_pre[...]
    mb = jnp.dot(h, proj[...], preferred_element_type=jnp.float32)
    d = h - mb
    var = jnp.dot(d * d, proj[...], preferred_element_type=jnp.float32)
    hn = d * lax.rsqrt(var + 1e-5)
    gate = jnp.dot(hn, wg[...], preferred_element_type=jnp.float32)
    val = jnp.dot(hn, wv[...], preferred_element_type=jnp.float32)
    z = jnp.maximum(gate, 0.0) * val
    y = jnp.dot(z, wpost_t[...], preferred_element_type=jnp.float32) + b_post[...]
    xx_out[...] = y * jnp.exp(scale[...])


def _tc_mlp(part, x_res, W_pre, b_pre, W_gate, W_value, W_post, b_post, scale):
    proj = jnp.kron(jnp.eye(G, dtype=jnp.float32),
                    jnp.ones((HIN, HIN), jnp.float32) / HIN)
    eye_g = jnp.eye(G, dtype=jnp.float32)
    wg = jnp.einsum('goi,gh->giho', W_gate, eye_g).reshape(G * HIN, G * HOUT)
    wv = jnp.einsum('goi,gh->giho', W_value, eye_g).reshape(G * HIN, G * HOUT)
    return pl.pallas_call(
        _mlp_body,
        out_shape=[
            jax.ShapeDtypeStruct((B, WIDTH), jnp.float32),
            jax.ShapeDtypeStruct((B, WIDTH), jnp.float32),
        ],
    )(part, x_res, W_pre.T, b_pre.reshape(1, WIDTH), proj, wg, wv,
      W_post.T, b_post.reshape(1, WIDTH), scale.reshape(1, WIDTH))


def kernel(x, x_res, batch, batch_size, W_pre, b_pre, W_gate, W_value, W_post,
           b_post, scale):
    del batch_size
    batch = batch.astype(jnp.int32)
    part = _sc_scatter(x, batch)
    xrn, xx = _tc_mlp(part, x_res, W_pre, b_pre, W_gate, W_value, W_post,
                      b_post, scale)
    out = _sc_gather(xx, batch)
    return (out, xrn)


# speedup vs baseline: 6.8311x; 1.0064x over previous
import functools

import jax
import jax.numpy as jnp
from jax import lax
from jax.experimental import pallas as pl
from jax.experimental.pallas import tpu as pltpu
from jax.experimental.pallas import tpu_sc as plsc

N, B, WIDTH = 100000, 1024, 128
G, HIN, WS = 8, 16, 2
HOUT = HIN * WS

NC, NS = 2, 16
NW = NC * NS
CH = 128
NFULL = N // CH
TAIL = N - NFULL * CH
KMAX = (NFULL + NW - 1) // NW
ROWS_PER_SUB = B // NS

SCH = 256
SNFULL = N // SCH
STAIL = N - SNFULL * SCH
SKMAX = (SNFULL + NW - 1) // NW
NBUF = 3


def _sc_mesh():
    return plsc.VectorSubcoreMesh(core_axis_name="c", subcore_axis_name="s")


def _sc_scatter(x, batch):
    @functools.partial(
        pl.kernel,
        out_type=jax.ShapeDtypeStruct((NC, B, WIDTH), jnp.float32),
        mesh=_sc_mesh(),
        scratch_types=[
            pltpu.VMEM((2 * NBUF, CH), jnp.int32),
            pltpu.VMEM((NBUF, SCH, WIDTH), jnp.float32),
            pltpu.VMEM((CH,), jnp.int32),
            pltpu.VMEM((CH, WIDTH), jnp.float32),
            pltpu.VMEM((TAIL,), jnp.int32),
            pltpu.VMEM((TAIL, WIDTH), jnp.float32),
            pltpu.VMEM((8, WIDTH), jnp.float32),
            pltpu.VMEM_SHARED((B, WIDTH), jnp.float32),
            pltpu.SemaphoreType.DMA,
            pltpu.SemaphoreType.DMA,
            pltpu.SemaphoreType.DMA,
        ],
    )
    def k(x_hbm, batch_hbm, part_hbm, idxb, rowsb, idx_ta, rows_ta, idx_tb,
          rows_tb, zbuf, acc_sh, isem, rsem, asem):
        cid = lax.axis_index("c")
        sid = lax.axis_index("s")
        wid = sid * NC + cid

        def issue_loads(c, b):
            base = pl.multiple_of(c * SCH, SCH)
            for j in range(SCH // CH):
                pltpu.async_copy(batch_hbm.at[pl.ds(base + j * CH, CH)],
                                 idxb.at[2 * b + j], isem)
            pltpu.async_copy(x_hbm.at[pl.ds(base, SCH)], rowsb.at[b], rsem)

        def wait_loads(c, b):
            base = pl.multiple_of(c * SCH, SCH)
            for j in range(SCH // CH):
                pltpu.make_async_copy(batch_hbm.at[pl.ds(base + j * CH, CH)],
                                      idxb.at[2 * b + j], isem).wait()
            pltpu.make_async_copy(x_hbm.at[pl.ds(base, SCH)], rowsb.at[b],
                                  rsem).wait()

        def wait_one_add(b):
            pltpu.make_async_copy(rowsb.at[b, pl.ds(0, CH)],
                                  acc_sh.at[idxb.at[2 * b]], asem).wait()

        issue_loads(jnp.int32(wid), 0)
        issue_loads(jnp.int32(wid + NW), 1)

        z16 = jnp.zeros((16,), jnp.float32)
        for r in range(8):
            for c8 in range(WIDTH // 16):
                zbuf[r, pl.ds(c8 * 16, 16)] = z16
        for t in range(ROWS_PER_SUB // 8):
            pltpu.sync_copy(zbuf, acc_sh.at[pl.ds(sid * ROWS_PER_SUB + t * 8, 8)])
        plsc.subcore_barrier()

        def chunk_body(kk, carry):
            c = wid + kk * NW
            b0 = lax.rem(kk, NBUF)

            @pl.when(c < SNFULL)
            def _():
                wait_loads(c, b0)
                @pl.when(kk >= 1)
                def _():
                    bp = lax.rem(kk + NBUF - 1, NBUF)
                    for j in range(SCH // CH):
                        wait_one_add(bp)
                for j in range(SCH // CH):
                    pltpu.async_copy(rowsb.at[b0, pl.ds(j * CH, CH)],
                                     acc_sh.at[idxb.at[2 * b0 + j]], asem,
                                     add=True)
                cpre = c + 2 * NW

                @pl.when(cpre < SNFULL)
                def _():
                    issue_loads(cpre, lax.rem(kk + 2, NBUF))

            return carry

        lax.fori_loop(0, SKMAX, chunk_body, 0)
        for j in range(SCH // CH):
            wait_one_add(0)

        @pl.when(wid == NW - 1)
        def _():
            tb = SNFULL * SCH
            pltpu.sync_copy(batch_hbm.at[pl.ds(tb, CH)], idx_ta)
            pltpu.sync_copy(x_hbm.at[pl.ds(tb, CH)], rows_ta)
            pltpu.sync_copy(rows_ta, acc_sh.at[idx_ta], add=True)
            pltpu.sync_copy(batch_hbm.at[pl.ds(tb + CH, TAIL)], idx_tb)
            pltpu.sync_copy(x_hbm.at[pl.ds(tb + CH, TAIL)], rows_tb)
            pltpu.sync_copy(rows_tb, acc_sh.at[idx_tb], add=True)

        plsc.subcore_barrier()
        pltpu.sync_copy(
            acc_sh.at[pl.ds(sid * ROWS_PER_SUB, ROWS_PER_SUB)],
            part_hbm.at[cid, pl.ds(sid * ROWS_PER_SUB, ROWS_PER_SUB)],
        )

    return k(x, batch)


def _sc_gather(xx, batch):
    @functools.partial(
        pl.kernel,
        out_type=jax.ShapeDtypeStruct((N, WIDTH), jnp.float32),
        mesh=_sc_mesh(),
        scratch_types=[
            pltpu.VMEM((2, CH), jnp.int32),
            pltpu.VMEM((2, CH, WIDTH), jnp.float32),
            pltpu.VMEM((TAIL,), jnp.int32),
            pltpu.VMEM((TAIL, WIDTH), jnp.float32),
            pltpu.VMEM_SHARED((B, WIDTH), jnp.float32),
            pltpu.SemaphoreType.DMA,
            pltpu.SemaphoreType.DMA,
            pltpu.SemaphoreType.DMA,
        ],
    )
    def k(xx_hbm, batch_hbm, out_hbm, idx2, rows2, idx_t, rows_t, xx_sh,
          isem, gsem, wsem):
        cid = lax.axis_index("c")
        sid = lax.axis_index("s")
        wid = sid * NC + cid

        base0 = pl.multiple_of(wid * CH, CH)
        pltpu.async_copy(batch_hbm.at[pl.ds(base0, CH)], idx2.at[0], isem)

        pltpu.sync_copy(
            xx_hbm.at[pl.ds(sid * ROWS_PER_SUB, ROWS_PER_SUB)],
            xx_sh.at[pl.ds(sid * ROWS_PER_SUB, ROWS_PER_SUB)],
        )
        plsc.subcore_barrier()

        def chunk_body(kk, carry):
            c = wid + kk * NW
            cn = c + NW
            b0 = lax.rem(kk, 2)
            b1 = lax.rem(kk + 1, 2)

            @pl.when(cn < NFULL)
            def _():
                basen = pl.multiple_of(cn * CH, CH)
                pltpu.async_copy(batch_hbm.at[pl.ds(basen, CH)], idx2.at[b1], isem)

            @pl.when(c < NFULL)
            def _():
                base = pl.multiple_of(c * CH, CH)
                @pl.when(kk >= 2)
                def _():
                    pltpu.make_async_copy(rows2.at[b0],
                                          out_hbm.at[pl.ds(base, CH)],
                                          wsem).wait()

                pltpu.make_async_copy(batch_hbm.at[pl.ds(base, CH)], idx2.at[b0],
                                      isem).wait()
                pltpu.async_copy(xx_sh.at[idx2.at[b0]], rows2.at[b0], gsem).wait()
                pltpu.async_copy(rows2.at[b0], out_hbm.at[pl.ds(base, CH)], wsem)

            return carry

        lax.fori_loop(0, KMAX, chunk_body, 0)

        for _ in range(2):
            pltpu.make_async_copy(rows2.at[0], out_hbm.at[pl.ds(0, CH)],
                                  wsem).wait()

        @pl.when(wid == NW - 1)
        def _():
            pltpu.sync_copy(batch_hbm.at[pl.ds(NFULL * CH, TAIL)], idx_t)
            pltpu.async_copy(xx_sh.at[idx_t], rows_t, gsem).wait()
            pltpu.sync_copy(rows_t, out_hbm.at[pl.ds(NFULL * CH, TAIL)])

    return k(xx, batch)


def _mlp_body(part, x_res, wpre_t, b_pre, proj, wg, wv, wpost_t, b_post, scale,
              xrn_out, xx_out):
    xr = part[0] + part[1] + x_res[...]
    xrn_out[...] = xr
    h = jnp.dot(xr, wpre_t[...], preferred_element_type=jnp.float32) + b_pre[...]
    mb = jnp.dot(h, proj[...], preferred_element_type=jnp.float32)
    d = h - mb
    var = jnp.dot(d * d, proj[...], preferred_element_type=jnp.float32)
    hn = d * lax.rsqrt(var + 1e-5)
    gate = jnp.dot(hn, wg[...], preferred_element_type=jnp.float32)
    val = jnp.dot(hn, wv[...], preferred_element_type=jnp.float32)
    z = jnp.maximum(gate, 0.0) * val
    y = jnp.dot(z, wpost_t[...], preferred_element_type=jnp.float32) + b_post[...]
    xx_out[...] = y * jnp.exp(scale[...])


def _tc_mlp(part, x_res, W_pre, b_pre, W_gate, W_value, W_post, b_post, scale):
    proj = jnp.kron(jnp.eye(G, dtype=jnp.float32),
                    jnp.ones((HIN, HIN), jnp.float32) / HIN)
    eye_g = jnp.eye(G, dtype=jnp.float32)
    wg = jnp.einsum('goi,gh->giho', W_gate, eye_g).reshape(G * HIN, G * HOUT)
    wv = jnp.einsum('goi,gh->giho', W_value, eye_g).reshape(G * HIN, G * HOUT)
    return pl.pallas_call(
        _mlp_body,
        out_shape=[
            jax.ShapeDtypeStruct((B, WIDTH), jnp.float32),
            jax.ShapeDtypeStruct((B, WIDTH), jnp.float32),
        ],
    )(part, x_res, W_pre.T, b_pre.reshape(1, WIDTH), proj, wg, wv,
      W_post.T, b_post.reshape(1, WIDTH), scale.reshape(1, WIDTH))


def kernel(x, x_res, batch, batch_size, W_pre, b_pre, W_gate, W_value, W_post,
           b_post, scale):
    del batch_size
    batch = batch.astype(jnp.int32)
    part = _sc_scatter(x, batch)
    xrn, xx = _tc_mlp(part, x_res, W_pre, b_pre, W_gate, W_value, W_post,
                      b_post, scale)
    out = _sc_gather(xx, batch)
    return (out, xrn)


# speedup vs baseline: 6.9779x; 1.0215x over previous
import functools

import jax
import jax.numpy as jnp
from jax import lax
from jax.experimental import pallas as pl
from jax.experimental.pallas import tpu as pltpu
from jax.experimental.pallas import tpu_sc as plsc

N, B, WIDTH = 100000, 1024, 128
G, HIN, WS = 8, 16, 2
HOUT = HIN * WS

NC, NS = 2, 16
NW = NC * NS
CH = 128
NFULL = N // CH
TAIL = N - NFULL * CH
KMAX = (NFULL + NW - 1) // NW
ROWS_PER_SUB = B // NS

SCH = 256
NBUF = 3

TCBLK = 1024
N_SC = 74 * TCBLK
TCNB = (N - N_SC + TCBLK - 1) // TCBLK
SNFULL = N_SC // SCH
SKMAX = (SNFULL + NW - 1) // NW


def _sc_mesh():
    return plsc.VectorSubcoreMesh(core_axis_name="c", subcore_axis_name="s")


def _sc_scatter(x, batch):
    @functools.partial(
        pl.kernel,
        out_type=jax.ShapeDtypeStruct((NC, B, WIDTH), jnp.float32),
        mesh=_sc_mesh(),
        scratch_types=[
            pltpu.VMEM((2 * NBUF, CH), jnp.int32),
            pltpu.VMEM((NBUF, SCH, WIDTH), jnp.float32),
            pltpu.VMEM((8, WIDTH), jnp.float32),
            pltpu.VMEM_SHARED((B, WIDTH), jnp.float32),
            pltpu.SemaphoreType.DMA,
            pltpu.SemaphoreType.DMA,
            pltpu.SemaphoreType.DMA,
        ],
    )
    def k(x_hbm, batch_hbm, part_hbm, idxb, rowsb, zbuf, acc_sh, isem, rsem,
          asem):
        cid = lax.axis_index("c")
        sid = lax.axis_index("s")
        wid = sid * NC + cid

        def issue_loads(c, b):
            base = pl.multiple_of(c * SCH, SCH)
            for j in range(SCH // CH):
                pltpu.async_copy(batch_hbm.at[pl.ds(base + j * CH, CH)],
                                 idxb.at[2 * b + j], isem)
            pltpu.async_copy(x_hbm.at[pl.ds(base, SCH)], rowsb.at[b], rsem)

        def wait_loads(c, b):
            base = pl.multiple_of(c * SCH, SCH)
            for j in range(SCH // CH):
                pltpu.make_async_copy(batch_hbm.at[pl.ds(base + j * CH, CH)],
                                      idxb.at[2 * b + j], isem).wait()
            pltpu.make_async_copy(x_hbm.at[pl.ds(base, SCH)], rowsb.at[b],
                                  rsem).wait()

        def wait_one_add(b):
            pltpu.make_async_copy(rowsb.at[b, pl.ds(0, CH)],
                                  acc_sh.at[idxb.at[2 * b]], asem).wait()

        issue_loads(jnp.int32(wid), 0)
        issue_loads(jnp.int32(wid + NW), 1)

        z16 = jnp.zeros((16,), jnp.float32)
        for r in range(8):
            for c8 in range(WIDTH // 16):
                zbuf[r, pl.ds(c8 * 16, 16)] = z16
        for t in range(ROWS_PER_SUB // 8):
            pltpu.sync_copy(zbuf, acc_sh.at[pl.ds(sid * ROWS_PER_SUB + t * 8, 8)])
        plsc.subcore_barrier()

        def chunk_body(kk, carry):
            c = wid + kk * NW
            b0 = lax.rem(kk, NBUF)

            @pl.when(c < SNFULL)
            def _():
                wait_loads(c, b0)
                @pl.when(kk >= 1)
                def _():
                    bp = lax.rem(kk + NBUF - 1, NBUF)
                    for j in range(SCH // CH):
                        wait_one_add(bp)
                for j in range(SCH // CH):
                    pltpu.async_copy(rowsb.at[b0, pl.ds(j * CH, CH)],
                                     acc_sh.at[idxb.at[2 * b0 + j]], asem,
                                     add=True)
                cpre = c + 2 * NW

                @pl.when(cpre < SNFULL)
                def _():
                    issue_loads(cpre, lax.rem(kk + 2, NBUF))

            return carry

        lax.fori_loop(0, SKMAX, chunk_body, 0)
        for j in range(SCH // CH):
            wait_one_add(0)

        plsc.subcore_barrier()
        pltpu.sync_copy(
            acc_sh.at[pl.ds(sid * ROWS_PER_SUB, ROWS_PER_SUB)],
            part_hbm.at[cid, pl.ds(sid * ROWS_PER_SUB, ROWS_PER_SUB)],
        )

    return k(x, batch)


def _sc_gather(xx, batch):
    @functools.partial(
        pl.kernel,
        out_type=jax.ShapeDtypeStruct((N, WIDTH), jnp.float32),
        mesh=_sc_mesh(),
        scratch_types=[
            pltpu.VMEM((2, CH), jnp.int32),
            pltpu.VMEM((2, CH, WIDTH), jnp.float32),
            pltpu.VMEM((TAIL,), jnp.int32),
            pltpu.VMEM((TAIL, WIDTH), jnp.float32),
            pltpu.VMEM_SHARED((B, WIDTH), jnp.float32),
            pltpu.SemaphoreType.DMA,
            pltpu.SemaphoreType.DMA,
            pltpu.SemaphoreType.DMA,
        ],
    )
    def k(xx_hbm, batch_hbm, out_hbm, idx2, rows2, idx_t, rows_t, xx_sh,
          isem, gsem, wsem):
        cid = lax.axis_index("c")
        sid = lax.axis_index("s")
        wid = sid * NC + cid

        base0 = pl.multiple_of(wid * CH, CH)
        pltpu.async_copy(batch_hbm.at[pl.ds(base0, CH)], idx2.at[0], isem)

        pltpu.sync_copy(
            xx_hbm.at[pl.ds(sid * ROWS_PER_SUB, ROWS_PER_SUB)],
            xx_sh.at[pl.ds(sid * ROWS_PER_SUB, ROWS_PER_SUB)],
        )
        plsc.subcore_barrier()

        def chunk_body(kk, carry):
            c = wid + kk * NW
            cn = c + NW
            b0 = lax.rem(kk, 2)
            b1 = lax.rem(kk + 1, 2)

            @pl.when(cn < NFULL)
            def _():
                basen = pl.multiple_of(cn * CH, CH)
                pltpu.async_copy(batch_hbm.at[pl.ds(basen, CH)], idx2.at[b1], isem)

            @pl.when(c < NFULL)
            def _():
                base = pl.multiple_of(c * CH, CH)
                @pl.when(kk >= 2)
                def _():
                    pltpu.make_async_copy(rows2.at[b0],
                                          out_hbm.at[pl.ds(base, CH)],
                                          wsem).wait()

                pltpu.make_async_copy(batch_hbm.at[pl.ds(base, CH)], idx2.at[b0],
                                      isem).wait()
                pltpu.async_copy(xx_sh.at[idx2.at[b0]], rows2.at[b0], gsem).wait()
                pltpu.async_copy(rows2.at[b0], out_hbm.at[pl.ds(base, CH)], wsem)

            return carry

        lax.fori_loop(0, KMAX, chunk_body, 0)

        for _ in range(2):
            pltpu.make_async_copy(rows2.at[0], out_hbm.at[pl.ds(0, CH)],
                                  wsem).wait()

        @pl.when(wid == NW - 1)
        def _():
            pltpu.sync_copy(batch_hbm.at[pl.ds(NFULL * CH, TAIL)], idx_t)
            pltpu.async_copy(xx_sh.at[idx_t], rows_t, gsem).wait()
            pltpu.sync_copy(rows_t, out_hbm.at[pl.ds(NFULL * CH, TAIL)])

    return k(xx, batch)


def _tc_scatter_body(ids_ref, x_ref, out_ref):
    k = pl.program_id(0)

    @pl.when(k == 0)
    def _():
        out_ref[...] = jnp.zeros_like(out_ref)

    base = (N_SC // TCBLK + k) * TCBLK
    ids = ids_ref[0, 0, :]
    rows = lax.broadcasted_iota(jnp.int32, (TCBLK, WIDTH), 0) + base
    xblk = jnp.where(rows < N, x_ref[...], 0.0)
    segs = lax.broadcasted_iota(jnp.int32, (B, TCBLK), 0)
    onehot = jnp.where(segs == ids[None, :], 1.0, 0.0)
    out_ref[...] += jnp.dot(onehot, xblk, preferred_element_type=jnp.float32)


def _tc_scatter(x, batch):
    npad = TCNB * TCBLK + N_SC - N
    batch_pad = jnp.concatenate(
        [batch, jnp.zeros((npad,), jnp.int32)]).reshape(-1, 1, TCBLK)
    return pl.pallas_call(
        _tc_scatter_body,
        grid=(TCNB,),
        in_specs=[
            pl.BlockSpec((1, 1, TCBLK), lambda k: (N_SC // TCBLK + k, 0, 0)),
            pl.BlockSpec((TCBLK, WIDTH), lambda k: (N_SC // TCBLK + k, 0)),
        ],
        out_specs=pl.BlockSpec((B, WIDTH), lambda k: (0, 0)),
        out_shape=jax.ShapeDtypeStruct((B, WIDTH), jnp.float32),
    )(batch_pad, x)


def _mlp_body(part, part_tc, x_res, wpre_t, b_pre, proj, wg, wv, wpost_t,
              b_post, scale, xrn_out, xx_out):
    xr = part[0] + part[1] + part_tc[...] + x_res[...]
    xrn_out[...] = xr
    h = jnp.dot(xr, wpre_t[...], preferred_element_type=jnp.float32) + b_pre[...]
    mb = jnp.dot(h, proj[...], preferred_element_type=jnp.float32)
    d = h - mb
    var = jnp.dot(d * d, proj[...], preferred_element_type=jnp.float32)
    hn = d * lax.rsqrt(var + 1e-5)
    gate = jnp.dot(hn, wg[...], preferred_element_type=jnp.float32)
    val = jnp.dot(hn, wv[...], preferred_element_type=jnp.float32)
    z = jnp.maximum(gate, 0.0) * val
    y = jnp.dot(z, wpost_t[...], preferred_element_type=jnp.float32) + b_post[...]
    xx_out[...] = y * jnp.exp(scale[...])


def _tc_mlp(part, part_tc, x_res, W_pre, b_pre, W_gate, W_value, W_post,
            b_post, scale):
    proj = jnp.kron(jnp.eye(G, dtype=jnp.float32),
                    jnp.ones((HIN, HIN), jnp.float32) / HIN)
    eye_g = jnp.eye(G, dtype=jnp.float32)
    wg = jnp.einsum('goi,gh->giho', W_gate, eye_g).reshape(G * HIN, G * HOUT)
    wv = jnp.einsum('goi,gh->giho', W_value, eye_g).reshape(G * HIN, G * HOUT)
    return pl.pallas_call(
        _mlp_body,
        out_shape=[
            jax.ShapeDtypeStruct((B, WIDTH), jnp.float32),
            jax.ShapeDtypeStruct((B, WIDTH), jnp.float32),
        ],
    )(part, part_tc, x_res, W_pre.T, b_pre.reshape(1, WIDTH), proj, wg, wv,
      W_post.T, b_post.reshape(1, WIDTH), scale.reshape(1, WIDTH))


def kernel(x, x_res, batch, batch_size, W_pre, b_pre, W_gate, W_value, W_post,
           b_post, scale):
    del batch_size
    batch = batch.astype(jnp.int32)
    part = _sc_scatter(x, batch)
    part_tc = _tc_scatter(x, batch)
    xrn, xx = _tc_mlp(part, part_tc, x_res, W_pre, b_pre, W_gate, W_value,
                      W_post, b_post, scale)
    out = _sc_gather(xx, batch)
    return (out, xrn)


# speedup vs baseline: 7.3192x; 1.0489x over previous
import functools

import jax
import jax.numpy as jnp
from jax import lax
from jax.experimental import pallas as pl
from jax.experimental.pallas import tpu as pltpu
from jax.experimental.pallas import tpu_sc as plsc

N, B, WIDTH = 100000, 1024, 128
G, HIN, WS = 8, 16, 2
HOUT = HIN * WS

NC, NS = 2, 16
NW = NC * NS
CH = 128
NFULL = N // CH
TAIL = N - NFULL * CH
KMAX = (NFULL + NW - 1) // NW
ROWS_PER_SUB = B // NS

SCH = 256
NBUF = 3

TCBLK = 1024
N_SC = 82 * TCBLK
TCNB = (N - N_SC + TCBLK - 1) // TCBLK
SNFULL = N_SC // SCH
SKMAX = (SNFULL + NW - 1) // NW


def _sc_mesh():
    return plsc.VectorSubcoreMesh(core_axis_name="c", subcore_axis_name="s")


def _sc_scatter(x, batch):
    @functools.partial(
        pl.kernel,
        out_type=jax.ShapeDtypeStruct((NC, B, WIDTH), jnp.float32),
        mesh=_sc_mesh(),
        scratch_types=[
            pltpu.VMEM((2 * NBUF, CH), jnp.int32),
            pltpu.VMEM((NBUF, SCH, WIDTH), jnp.float32),
            pltpu.VMEM((8, WIDTH), jnp.float32),
            pltpu.VMEM_SHARED((B, WIDTH), jnp.float32),
            pltpu.SemaphoreType.DMA,
            pltpu.SemaphoreType.DMA,
            pltpu.SemaphoreType.DMA,
        ],
    )
    def k(x_hbm, batch_hbm, part_hbm, idxb, rowsb, zbuf, acc_sh, isem, rsem,
          asem):
        cid = lax.axis_index("c")
        sid = lax.axis_index("s")
        wid = sid * NC + cid

        def issue_loads(c, b):
            base = pl.multiple_of(c * SCH, SCH)
            for j in range(SCH // CH):
                pltpu.async_copy(batch_hbm.at[pl.ds(base + j * CH, CH)],
                                 idxb.at[2 * b + j], isem)
            pltpu.async_copy(x_hbm.at[pl.ds(base, SCH)], rowsb.at[b], rsem)

        def wait_loads(c, b):
            base = pl.multiple_of(c * SCH, SCH)
            for j in range(SCH // CH):
                pltpu.make_async_copy(batch_hbm.at[pl.ds(base + j * CH, CH)],
                                      idxb.at[2 * b + j], isem).wait()
            pltpu.make_async_copy(x_hbm.at[pl.ds(base, SCH)], rowsb.at[b],
                                  rsem).wait()

        def wait_one_add(b):
            pltpu.make_async_copy(rowsb.at[b, pl.ds(0, CH)],
                                  acc_sh.at[idxb.at[2 * b]], asem).wait()

        issue_loads(jnp.int32(wid), 0)
        issue_loads(jnp.int32(wid + NW), 1)

        z16 = jnp.zeros((16,), jnp.float32)
        for r in range(8):
            for c8 in range(WIDTH // 16):
                zbuf[r, pl.ds(c8 * 16, 16)] = z16
        for t in range(ROWS_PER_SUB // 8):
            pltpu.sync_copy(zbuf, acc_sh.at[pl.ds(sid * ROWS_PER_SUB + t * 8, 8)])
        plsc.subcore_barrier()

        def chunk_body(kk, carry):
            c = wid + kk * NW
            b0 = lax.rem(kk, NBUF)

            @pl.when(c < SNFULL)
            def _():
                wait_loads(c, b0)
                @pl.when(kk >= 1)
                def _():
                    bp = lax.rem(kk + NBUF - 1, NBUF)
                    for j in range(SCH // CH):
                        wait_one_add(bp)
                for j in range(SCH // CH):
                    pltpu.async_copy(rowsb.at[b0, pl.ds(j * CH, CH)],
                                     acc_sh.at[idxb.at[2 * b0 + j]], asem,
                                     add=True)
                cpre = c + 2 * NW

                @pl.when(cpre < SNFULL)
                def _():
                    issue_loads(cpre, lax.rem(kk + 2, NBUF))

            return carry

        lax.fori_loop(0, SKMAX, chunk_body, 0)
        for j in range(SCH // CH):
            wait_one_add(0)

        plsc.subcore_barrier()
        pltpu.sync_copy(
            acc_sh.at[pl.ds(sid * ROWS_PER_SUB, ROWS_PER_SUB)],
            part_hbm.at[cid, pl.ds(sid * ROWS_PER_SUB, ROWS_PER_SUB)],
        )

    return k(x, batch)


def _sc_gather(xx, batch):
    @functools.partial(
        pl.kernel,
        out_type=jax.ShapeDtypeStruct((N, WIDTH), jnp.float32),
        mesh=_sc_mesh(),
        scratch_types=[
            pltpu.VMEM((2, CH), jnp.int32),
            pltpu.VMEM((2, CH, WIDTH), jnp.float32),
            pltpu.VMEM((TAIL,), jnp.int32),
            pltpu.VMEM((TAIL, WIDTH), jnp.float32),
            pltpu.VMEM_SHARED((B, WIDTH), jnp.float32),
            pltpu.SemaphoreType.DMA,
            pltpu.SemaphoreType.DMA,
            pltpu.SemaphoreType.DMA,
        ],
    )
    def k(xx_hbm, batch_hbm, out_hbm, idx2, rows2, idx_t, rows_t, xx_sh,
          isem, gsem, wsem):
        cid = lax.axis_index("c")
        sid = lax.axis_index("s")
        wid = sid * NC + cid

        base0 = pl.multiple_of(wid * CH, CH)
        pltpu.async_copy(batch_hbm.at[pl.ds(base0, CH)], idx2.at[0], isem)

        pltpu.sync_copy(
            xx_hbm.at[pl.ds(sid * ROWS_PER_SUB, ROWS_PER_SUB)],
            xx_sh.at[pl.ds(sid * ROWS_PER_SUB, ROWS_PER_SUB)],
        )
        plsc.subcore_barrier()

        def chunk_body(kk, carry):
            c = wid + kk * NW
            cn = c + NW
            b0 = lax.rem(kk, 2)
            b1 = lax.rem(kk + 1, 2)

            @pl.when(cn < NFULL)
            def _():
                basen = pl.multiple_of(cn * CH, CH)
                pltpu.async_copy(batch_hbm.at[pl.ds(basen, CH)], idx2.at[b1], isem)

            @pl.when(c < NFULL)
            def _():
                base = pl.multiple_of(c * CH, CH)
                @pl.when(kk >= 2)
                def _():
                    pltpu.make_async_copy(rows2.at[b0],
                                          out_hbm.at[pl.ds(base, CH)],
                                          wsem).wait()

                pltpu.make_async_copy(batch_hbm.at[pl.ds(base, CH)], idx2.at[b0],
                                      isem).wait()
                pltpu.async_copy(xx_sh.at[idx2.at[b0]], rows2.at[b0], gsem).wait()
                pltpu.async_copy(rows2.at[b0], out_hbm.at[pl.ds(base, CH)], wsem)

            return carry

        lax.fori_loop(0, KMAX, chunk_body, 0)

        for _ in range(2):
            pltpu.make_async_copy(rows2.at[0], out_hbm.at[pl.ds(0, CH)],
                                  wsem).wait()

        @pl.when(wid == NW - 1)
        def _():
            pltpu.sync_copy(batch_hbm.at[pl.ds(NFULL * CH, TAIL)], idx_t)
            pltpu.async_copy(xx_sh.at[idx_t], rows_t, gsem).wait()
            pltpu.sync_copy(rows_t, out_hbm.at[pl.ds(NFULL * CH, TAIL)])

    return k(xx, batch)


def _tc_scatter_body(ids_ref, x_ref, out_ref):
    k = pl.program_id(0)

    @pl.when(k == 0)
    def _():
        out_ref[...] = jnp.zeros_like(out_ref)

    base = (N_SC // TCBLK + k) * TCBLK
    ids = ids_ref[0, 0, :]
    rows = lax.broadcasted_iota(jnp.int32, (TCBLK, WIDTH), 0) + base
    xblk = jnp.where(rows < N, x_ref[...], 0.0)
    segs = lax.broadcasted_iota(jnp.int32, (B, TCBLK), 0)
    onehot = jnp.where(segs == ids[None, :], 1.0, 0.0)
    out_ref[...] += jnp.dot(onehot, xblk, preferred_element_type=jnp.float32)


def _tc_scatter(x, batch):
    npad = TCNB * TCBLK + N_SC - N
    batch_pad = jnp.concatenate(
        [batch, jnp.zeros((npad,), jnp.int32)]).reshape(-1, 1, TCBLK)
    return pl.pallas_call(
        _tc_scatter_body,
        grid=(TCNB,),
        in_specs=[
            pl.BlockSpec((1, 1, TCBLK), lambda k: (N_SC // TCBLK + k, 0, 0)),
            pl.BlockSpec((TCBLK, WIDTH), lambda k: (N_SC // TCBLK + k, 0)),
        ],
        out_specs=pl.BlockSpec((B, WIDTH), lambda k: (0, 0)),
        out_shape=jax.ShapeDtypeStruct((B, WIDTH), jnp.float32),
    )(batch_pad, x)


def _mlp_body(part, part_tc, x_res, wpre_t, b_pre, proj, wg, wv, wpost_t,
              b_post, scale, xrn_out, xx_out):
    xr = part[0] + part[1] + part_tc[...] + x_res[...]
    xrn_out[...] = xr
    h = jnp.dot(xr, wpre_t[...], preferred_element_type=jnp.float32) + b_pre[...]
    mb = jnp.dot(h, proj[...], preferred_element_type=jnp.float32)
    d = h - mb
    var = jnp.dot(d * d, proj[...], preferred_element_type=jnp.float32)
    hn = d * lax.rsqrt(var + 1e-5)
    gate = jnp.dot(hn, wg[...], preferred_element_type=jnp.float32)
    val = jnp.dot(hn, wv[...], preferred_element_type=jnp.float32)
    z = jnp.maximum(gate, 0.0) * val
    y = jnp.dot(z, wpost_t[...], preferred_element_type=jnp.float32) + b_post[...]
    xx_out[...] = y * jnp.exp(scale[...])


def _tc_mlp(part, part_tc, x_res, W_pre, b_pre, W_gate, W_value, W_post,
            b_post, scale):
    proj = jnp.kron(jnp.eye(G, dtype=jnp.float32),
                    jnp.ones((HIN, HIN), jnp.float32) / HIN)
    eye_g = jnp.eye(G, dtype=jnp.float32)
    wg = jnp.einsum('goi,gh->giho', W_gate, eye_g).reshape(G * HIN, G * HOUT)
    wv = jnp.einsum('goi,gh->giho', W_value, eye_g).reshape(G * HIN, G * HOUT)
    return pl.pallas_call(
        _mlp_body,
        out_shape=[
            jax.ShapeDtypeStruct((B, WIDTH), jnp.float32),
            jax.ShapeDtypeStruct((B, WIDTH), jnp.float32),
        ],
    )(part, part_tc, x_res, W_pre.T, b_pre.reshape(1, WIDTH), proj, wg, wv,
      W_post.T, b_post.reshape(1, WIDTH), scale.reshape(1, WIDTH))


def kernel(x, x_res, batch, batch_size, W_pre, b_pre, W_gate, W_value, W_post,
           b_post, scale):
    del batch_size
    batch = batch.astype(jnp.int32)
    part = _sc_scatter(x, batch)
    part_tc = _tc_scatter(x, batch)
    xrn, xx = _tc_mlp(part, part_tc, x_res, W_pre, b_pre, W_gate, W_value,
                      W_post, b_post, scale)
    out = _sc_gather(xx, batch)
    return (out, xrn)


# speedup vs baseline: 7.4036x; 1.0115x over previous
import functools

import jax
import jax.numpy as jnp
from jax import lax
from jax.experimental import pallas as pl
from jax.experimental.pallas import tpu as pltpu
from jax.experimental.pallas import tpu_sc as plsc

N, B, WIDTH = 100000, 1024, 128
G, HIN, WS = 8, 16, 2
HOUT = HIN * WS

NC, NS = 2, 16
NW = NC * NS
CH = 128
NFULL = N // CH
TAIL = N - NFULL * CH
KMAX = (NFULL + NW - 1) // NW
ROWS_PER_SUB = B // NS

SCH = 256
NBUF = 3

TCBLK = 1024
N_SC = 78 * TCBLK
TCNB = (N - N_SC + TCBLK - 1) // TCBLK
SNFULL = N_SC // SCH
SKMAX = (SNFULL + NW - 1) // NW


def _sc_mesh():
    return plsc.VectorSubcoreMesh(core_axis_name="c", subcore_axis_name="s")


def _sc_scatter(x, batch):
    @functools.partial(
        pl.kernel,
        out_type=jax.ShapeDtypeStruct((NC, B, WIDTH), jnp.float32),
        mesh=_sc_mesh(),
        scratch_types=[
            pltpu.VMEM((2 * NBUF, CH), jnp.int32),
            pltpu.VMEM((NBUF, SCH, WIDTH), jnp.float32),
            pltpu.VMEM((8, WIDTH), jnp.float32),
            pltpu.VMEM_SHARED((B, WIDTH), jnp.float32),
            pltpu.SemaphoreType.DMA,
            pltpu.SemaphoreType.DMA,
            pltpu.SemaphoreType.DMA,
        ],
    )
    def k(x_hbm, batch_hbm, part_hbm, idxb, rowsb, zbuf, acc_sh, isem, rsem,
          asem):
        cid = lax.axis_index("c")
        sid = lax.axis_index("s")
        wid = sid * NC + cid

        def issue_loads(c, b):
            base = pl.multiple_of(c * SCH, SCH)
            for j in range(SCH // CH):
                pltpu.async_copy(batch_hbm.at[pl.ds(base + j * CH, CH)],
                                 idxb.at[2 * b + j], isem)
            pltpu.async_copy(x_hbm.at[pl.ds(base, SCH)], rowsb.at[b], rsem)

        def wait_loads(c, b):
            base = pl.multiple_of(c * SCH, SCH)
            for j in range(SCH // CH):
                pltpu.make_async_copy(batch_hbm.at[pl.ds(base + j * CH, CH)],
                                      idxb.at[2 * b + j], isem).wait()
            pltpu.make_async_copy(x_hbm.at[pl.ds(base, SCH)], rowsb.at[b],
                                  rsem).wait()

        def wait_one_add(b):
            pltpu.make_async_copy(rowsb.at[b, pl.ds(0, CH)],
                                  acc_sh.at[idxb.at[2 * b]], asem).wait()

        issue_loads(jnp.int32(wid), 0)
        issue_loads(jnp.int32(wid + NW), 1)

        z16 = jnp.zeros((16,), jnp.float32)
        for r in range(8):
            for c8 in range(WIDTH // 16):
                zbuf[r, pl.ds(c8 * 16, 16)] = z16
        for t in range(ROWS_PER_SUB // 8):
            pltpu.sync_copy(zbuf, acc_sh.at[pl.ds(sid * ROWS_PER_SUB + t * 8, 8)])
        plsc.subcore_barrier()

        def chunk_body(kk, carry):
            c = wid + kk * NW
            b0 = lax.rem(kk, NBUF)

            @pl.when(c < SNFULL)
            def _():
                wait_loads(c, b0)
                @pl.when(kk >= 1)
                def _():
                    bp = lax.rem(kk + NBUF - 1, NBUF)
                    for j in range(SCH // CH):
                        wait_one_add(bp)
                for j in range(SCH // CH):
                    pltpu.async_copy(rowsb.at[b0, pl.ds(j * CH, CH)],
                                     acc_sh.at[idxb.at[2 * b0 + j]], asem,
                                     add=True)
                cpre = c + 2 * NW

                @pl.when(cpre < SNFULL)
                def _():
                    issue_loads(cpre, lax.rem(kk + 2, NBUF))

            return carry

        lax.fori_loop(0, SKMAX, chunk_body, 0)
        for j in range(SCH // CH):
            wait_one_add(0)

        plsc.subcore_barrier()
        pltpu.sync_copy(
            acc_sh.at[pl.ds(sid * ROWS_PER_SUB, ROWS_PER_SUB)],
            part_hbm.at[cid, pl.ds(sid * ROWS_PER_SUB, ROWS_PER_SUB)],
        )

    return k(x, batch)


def _sc_gather(xx, batch):
    @functools.partial(
        pl.kernel,
        out_type=jax.ShapeDtypeStruct((N, WIDTH), jnp.float32),
        mesh=_sc_mesh(),
        scratch_types=[
            pltpu.VMEM((2, CH), jnp.int32),
            pltpu.VMEM((2, CH, WIDTH), jnp.float32),
            pltpu.VMEM((TAIL,), jnp.int32),
            pltpu.VMEM((TAIL, WIDTH), jnp.float32),
            pltpu.VMEM_SHARED((B, WIDTH), jnp.float32),
            pltpu.SemaphoreType.DMA,
            pltpu.SemaphoreType.DMA,
            pltpu.SemaphoreType.DMA,
        ],
    )
    def k(xx_hbm, batch_hbm, out_hbm, idx2, rows2, idx_t, rows_t, xx_sh,
          isem, gsem, wsem):
        cid = lax.axis_index("c")
        sid = lax.axis_index("s")
        wid = sid * NC + cid

        base0 = pl.multiple_of(wid * CH, CH)
        pltpu.async_copy(batch_hbm.at[pl.ds(base0, CH)], idx2.at[0], isem)

        pltpu.sync_copy(
            xx_hbm.at[pl.ds(sid * ROWS_PER_SUB, ROWS_PER_SUB)],
            xx_sh.at[pl.ds(sid * ROWS_PER_SUB, ROWS_PER_SUB)],
        )
        plsc.subcore_barrier()

        def chunk_body(kk, carry):
            c = wid + kk * NW
            cn = c + NW
            b0 = lax.rem(kk, 2)
            b1 = lax.rem(kk + 1, 2)

            @pl.when(cn < NFULL)
            def _():
                basen = pl.multiple_of(cn * CH, CH)
                pltpu.async_copy(batch_hbm.at[pl.ds(basen, CH)], idx2.at[b1], isem)

            @pl.when(c < NFULL)
            def _():
                base = pl.multiple_of(c * CH, CH)
                @pl.when(kk >= 2)
                def _():
                    pltpu.make_async_copy(rows2.at[b0],
                                          out_hbm.at[pl.ds(base, CH)],
                                          wsem).wait()

                pltpu.make_async_copy(batch_hbm.at[pl.ds(base, CH)], idx2.at[b0],
                                      isem).wait()
                pltpu.async_copy(xx_sh.at[idx2.at[b0]], rows2.at[b0], gsem).wait()
                pltpu.async_copy(rows2.at[b0], out_hbm.at[pl.ds(base, CH)], wsem)

            return carry

        lax.fori_loop(0, KMAX, chunk_body, 0)

        for _ in range(2):
            pltpu.make_async_copy(rows2.at[0], out_hbm.at[pl.ds(0, CH)],
                                  wsem).wait()

        @pl.when(wid == NW - 1)
        def _():
            pltpu.sync_copy(batch_hbm.at[pl.ds(NFULL * CH, TAIL)], idx_t)
            pltpu.async_copy(xx_sh.at[idx_t], rows_t, gsem).wait()
            pltpu.sync_copy(rows_t, out_hbm.at[pl.ds(NFULL * CH, TAIL)])

    return k(xx, batch)


def _tc_scatter_body(ids_ref, x_ref, out_ref):
    k = pl.program_id(0)

    @pl.when(k == 0)
    def _():
        out_ref[...] = jnp.zeros_like(out_ref)

    base = (N_SC // TCBLK + k) * TCBLK
    ids = ids_ref[0, 0, :]
    rows = lax.broadcasted_iota(jnp.int32, (TCBLK, WIDTH), 0) + base
    xblk = jnp.where(rows < N, x_ref[...], 0.0)
    segs = lax.broadcasted_iota(jnp.int32, (B, TCBLK), 0)
    onehot = jnp.where(segs == ids[None, :], 1.0, 0.0)
    out_ref[...] += jnp.dot(onehot, xblk, preferred_element_type=jnp.float32)


def _tc_scatter(x, batch):
    npad = TCNB * TCBLK + N_SC - N
    batch_pad = jnp.concatenate(
        [batch, jnp.zeros((npad,), jnp.int32)]).reshape(-1, 1, TCBLK)
    return pl.pallas_call(
        _tc_scatter_body,
        grid=(TCNB,),
        in_specs=[
            pl.BlockSpec((1, 1, TCBLK), lambda k: (N_SC // TCBLK + k, 0, 0)),
            pl.BlockSpec((TCBLK, WIDTH), lambda k: (N_SC // TCBLK + k, 0)),
        ],
        out_specs=pl.BlockSpec((B, WIDTH), lambda k: (0, 0)),
        out_shape=jax.ShapeDtypeStruct((B, WIDTH), jnp.float32),
    )(batch_pad, x)


def _mlp_body(part, part_tc, x_res, wpre_t, b_pre, proj, wg, wv, wpost_t,
              b_post, scale, xrn_out, xx_out):
    xr = part[0] + part[1] + part_tc[...] + x_res[...]
    xrn_out[...] = xr
    h = jnp.dot(xr, wpre_t[...], preferred_element_type=jnp.float32) + b_pre[...]
    mb = jnp.dot(h, proj[...], preferred_element_type=jnp.float32)
    d = h - mb
    var = jnp.dot(d * d, proj[...], preferred_element_type=jnp.float32)
    hn = d * lax.rsqrt(var + 1e-5)
    gate = jnp.dot(hn, wg[...], preferred_element_type=jnp.float32)
    val = jnp.dot(hn, wv[...], preferred_element_type=jnp.float32)
    z = jnp.maximum(gate, 0.0) * val
    y = jnp.dot(z, wpost_t[...], preferred_element_type=jnp.float32) + b_post[...]
    xx_out[...] = y * jnp.exp(scale[...])


def _tc_mlp(part, part_tc, x_res, W_pre, b_pre, W_gate, W_value, W_post,
            b_post, scale):
    proj = jnp.kron(jnp.eye(G, dtype=jnp.float32),
                    jnp.ones((HIN, HIN), jnp.float32) / HIN)
    eye_g = jnp.eye(G, dtype=jnp.float32)
    wg = jnp.einsum('goi,gh->giho', W_gate, eye_g).reshape(G * HIN, G * HOUT)
    wv = jnp.einsum('goi,gh->giho', W_value, eye_g).reshape(G * HIN, G * HOUT)
    return pl.pallas_call(
        _mlp_body,
        out_shape=[
            jax.ShapeDtypeStruct((B, WIDTH), jnp.float32),
            jax.ShapeDtypeStruct((B, WIDTH), jnp.float32),
        ],
    )(part, part_tc, x_res, W_pre.T, b_pre.reshape(1, WIDTH), proj, wg, wv,
      W_post.T, b_post.reshape(1, WIDTH), scale.reshape(1, WIDTH))


def kernel(x, x_res, batch, batch_size, W_pre, b_pre, W_gate, W_value, W_post,
           b_post, scale):
    del batch_size
    batch = batch.astype(jnp.int32)
    part = _sc_scatter(x, batch)
    part_tc = _tc_scatter(x, batch)
    xrn, xx = _tc_mlp(part, part_tc, x_res, W_pre, b_pre, W_gate, W_value,
                      W_post, b_post, scale)
    out = _sc_gather(xx, batch)
    return (out, xrn)


# speedup vs baseline: 7.4061x; 1.0003x over previous
import functools

import jax
import jax.numpy as jnp
from jax import lax
from jax.experimental import pallas as pl
from jax.experimental.pallas import tpu as pltpu
from jax.experimental.pallas import tpu_sc as plsc

N, B, WIDTH = 100000, 1024, 128
G, HIN, WS = 8, 16, 2
HOUT = HIN * WS

NC, NS = 2, 16
NW = NC * NS
CH = 128
NFULL = N // CH
TAIL = N - NFULL * CH
KMAX = (NFULL + NW - 1) // NW
ROWS_PER_SUB = B // NS

SCH = 256
NBUF = 3

TCBLK = 1024
N_SC = 78 * TCBLK
TCNB = (N - N_SC + TCBLK - 1) // TCBLK
SNFULL = N_SC // SCH
SKMAX = (SNFULL + NW - 1) // NW


def _sc_mesh():
    return plsc.VectorSubcoreMesh(core_axis_name="c", subcore_axis_name="s")


def _sc_scatter(x, batch):
    @functools.partial(
        pl.kernel,
        out_type=jax.ShapeDtypeStruct((NC, B, WIDTH), jnp.float32),
        mesh=_sc_mesh(),
        scratch_types=[
            pltpu.VMEM((2 * NBUF, CH), jnp.int32),
            pltpu.VMEM((NBUF, SCH, WIDTH), jnp.float32),
            pltpu.VMEM((8, WIDTH), jnp.float32),
            pltpu.VMEM_SHARED((B, WIDTH), jnp.float32),
            pltpu.SemaphoreType.DMA,
            pltpu.SemaphoreType.DMA,
            pltpu.SemaphoreType.DMA,
            pltpu.SemaphoreType.DMA,
        ],
    )
    def k(x_hbm, batch_hbm, part_hbm, idxb, rowsb, zbuf, acc_sh, isem, rsem,
          asem, zsem):
        cid = lax.axis_index("c")
        sid = lax.axis_index("s")
        wid = sid * NC + cid

        def issue_loads(c, b):
            base = pl.multiple_of(c * SCH, SCH)
            for j in range(SCH // CH):
                pltpu.async_copy(batch_hbm.at[pl.ds(base + j * CH, CH)],
                                 idxb.at[2 * b + j], isem)
            pltpu.async_copy(x_hbm.at[pl.ds(base, SCH)], rowsb.at[b], rsem)

        def wait_loads(c, b):
            base = pl.multiple_of(c * SCH, SCH)
            for j in range(SCH // CH):
                pltpu.make_async_copy(batch_hbm.at[pl.ds(base + j * CH, CH)],
                                      idxb.at[2 * b + j], isem).wait()
            pltpu.make_async_copy(x_hbm.at[pl.ds(base, SCH)], rowsb.at[b],
                                  rsem).wait()

        def wait_one_add(b):
            pltpu.make_async_copy(rowsb.at[b, pl.ds(0, CH)],
                                  acc_sh.at[idxb.at[2 * b]], asem).wait()

        issue_loads(jnp.int32(wid), 0)
        issue_loads(jnp.int32(wid + NW), 1)

        z16 = jnp.zeros((16,), jnp.float32)
        for r in range(8):
            for c8 in range(WIDTH // 16):
                zbuf[r, pl.ds(c8 * 16, 16)] = z16
        for t in range(ROWS_PER_SUB // 8):
            pltpu.async_copy(zbuf, acc_sh.at[pl.ds(sid * ROWS_PER_SUB + t * 8, 8)],
                             zsem)
        for t in range(ROWS_PER_SUB // 8):
            pltpu.make_async_copy(
                zbuf, acc_sh.at[pl.ds(sid * ROWS_PER_SUB + t * 8, 8)],
                zsem).wait()
        plsc.subcore_barrier()

        def chunk_body(kk, carry):
            c = wid + kk * NW
            b0 = lax.rem(kk, NBUF)

            @pl.when(c < SNFULL)
            def _():
                wait_loads(c, b0)
                @pl.when(kk >= 1)
                def _():
                    bp = lax.rem(kk + NBUF - 1, NBUF)
                    for j in range(SCH // CH):
                        wait_one_add(bp)
                for j in range(SCH // CH):
                    pltpu.async_copy(rowsb.at[b0, pl.ds(j * CH, CH)],
                                     acc_sh.at[idxb.at[2 * b0 + j]], asem,
                                     add=True)
                cpre = c + 2 * NW

                @pl.when(cpre < SNFULL)
                def _():
                    issue_loads(cpre, lax.rem(kk + 2, NBUF))

            return carry

        lax.fori_loop(0, SKMAX, chunk_body, 0)
        for j in range(SCH // CH):
            wait_one_add(0)

        plsc.subcore_barrier()
        pltpu.sync_copy(
            acc_sh.at[pl.ds(sid * ROWS_PER_SUB, ROWS_PER_SUB)],
            part_hbm.at[cid, pl.ds(sid * ROWS_PER_SUB, ROWS_PER_SUB)],
        )

    return k(x, batch)


def _sc_gather(xx, batch):
    @functools.partial(
        pl.kernel,
        out_type=jax.ShapeDtypeStruct((N, WIDTH), jnp.float32),
        mesh=_sc_mesh(),
        scratch_types=[
            pltpu.VMEM((2, CH), jnp.int32),
            pltpu.VMEM((2, CH, WIDTH), jnp.float32),
            pltpu.VMEM((TAIL,), jnp.int32),
            pltpu.VMEM((TAIL, WIDTH), jnp.float32),
            pltpu.VMEM_SHARED((B, WIDTH), jnp.float32),
            pltpu.SemaphoreType.DMA,
            pltpu.SemaphoreType.DMA,
            pltpu.SemaphoreType.DMA,
        ],
    )
    def k(xx_hbm, batch_hbm, out_hbm, idx2, rows2, idx_t, rows_t, xx_sh,
          isem, gsem, wsem):
        cid = lax.axis_index("c")
        sid = lax.axis_index("s")
        wid = sid * NC + cid

        base0 = pl.multiple_of(wid * CH, CH)
        pltpu.async_copy(batch_hbm.at[pl.ds(base0, CH)], idx2.at[0], isem)

        pltpu.sync_copy(
            xx_hbm.at[pl.ds(sid * ROWS_PER_SUB, ROWS_PER_SUB)],
            xx_sh.at[pl.ds(sid * ROWS_PER_SUB, ROWS_PER_SUB)],
        )
        plsc.subcore_barrier()

        def chunk_body(kk, carry):
            c = wid + kk * NW
            cn = c + NW
            b0 = lax.rem(kk, 2)
            b1 = lax.rem(kk + 1, 2)

            @pl.when(cn < NFULL)
            def _():
                basen = pl.multiple_of(cn * CH, CH)
                pltpu.async_copy(batch_hbm.at[pl.ds(basen, CH)], idx2.at[b1], isem)

            @pl.when(c < NFULL)
            def _():
                base = pl.multiple_of(c * CH, CH)
                @pl.when(kk >= 2)
                def _():
                    pltpu.make_async_copy(rows2.at[b0],
                                          out_hbm.at[pl.ds(base, CH)],
                                          wsem).wait()

                pltpu.make_async_copy(batch_hbm.at[pl.ds(base, CH)], idx2.at[b0],
                                      isem).wait()
                pltpu.async_copy(xx_sh.at[idx2.at[b0]], rows2.at[b0], gsem).wait()
                pltpu.async_copy(rows2.at[b0], out_hbm.at[pl.ds(base, CH)], wsem)

            return carry

        lax.fori_loop(0, KMAX, chunk_body, 0)

        for _ in range(2):
            pltpu.make_async_copy(rows2.at[0], out_hbm.at[pl.ds(0, CH)],
                                  wsem).wait()

        @pl.when(wid == NW - 1)
        def _():
            pltpu.sync_copy(batch_hbm.at[pl.ds(NFULL * CH, TAIL)], idx_t)
            pltpu.async_copy(xx_sh.at[idx_t], rows_t, gsem).wait()
            pltpu.sync_copy(rows_t, out_hbm.at[pl.ds(NFULL * CH, TAIL)])

    return k(xx, batch)


def _tc_scatter_body(ids_ref, x_ref, out_ref):
    k = pl.program_id(0)

    @pl.when(k == 0)
    def _():
        out_ref[...] = jnp.zeros_like(out_ref)

    base = (N_SC // TCBLK + k) * TCBLK
    ids = ids_ref[0, 0, :]
    rows = lax.broadcasted_iota(jnp.int32, (TCBLK, WIDTH), 0) + base
    xblk = jnp.where(rows < N, x_ref[...], 0.0)
    segs = lax.broadcasted_iota(jnp.int32, (B, TCBLK), 0)
    onehot = jnp.where(segs == ids[None, :], 1.0, 0.0)
    out_ref[...] += jnp.dot(onehot, xblk, preferred_element_type=jnp.float32)


def _tc_scatter(x, batch):
    npad = TCNB * TCBLK + N_SC - N
    batch_pad = jnp.concatenate(
        [batch, jnp.zeros((npad,), jnp.int32)]).reshape(-1, 1, TCBLK)
    return pl.pallas_call(
        _tc_scatter_body,
        grid=(TCNB,),
        in_specs=[
            pl.BlockSpec((1, 1, TCBLK), lambda k: (N_SC // TCBLK + k, 0, 0)),
            pl.BlockSpec((TCBLK, WIDTH), lambda k: (N_SC // TCBLK + k, 0)),
        ],
        out_specs=pl.BlockSpec((B, WIDTH), lambda k: (0, 0)),
        out_shape=jax.ShapeDtypeStruct((B, WIDTH), jnp.float32),
    )(batch_pad, x)


def _mlp_body(part, part_tc, x_res, wpre_t, b_pre, proj, wg, wv, wpost_t,
              b_post, scale, xrn_out, xx_out):
    xr = part[0] + part[1] + part_tc[...] + x_res[...]
    xrn_out[...] = xr
    h = jnp.dot(xr, wpre_t[...], preferred_element_type=jnp.float32) + b_pre[...]
    mb = jnp.dot(h, proj[...], preferred_element_type=jnp.float32)
    d = h - mb
    var = jnp.dot(d * d, proj[...], preferred_element_type=jnp.float32)
    hn = d * lax.rsqrt(var + 1e-5)
    gate = jnp.dot(hn, wg[...], preferred_element_type=jnp.float32)
    val = jnp.dot(hn, wv[...], preferred_element_type=jnp.float32)
    z = jnp.maximum(gate, 0.0) * val
    y = jnp.dot(z, wpost_t[...], preferred_element_type=jnp.float32) + b_post[...]
    xx_out[...] = y * jnp.exp(scale[...])


def _tc_mlp(part, part_tc, x_res, W_pre, b_pre, W_gate, W_value, W_post,
            b_post, scale):
    proj = jnp.kron(jnp.eye(G, dtype=jnp.float32),
                    jnp.ones((HIN, HIN), jnp.float32) / HIN)
    eye_g = jnp.eye(G, dtype=jnp.float32)
    wg = jnp.einsum('goi,gh->giho', W_gate, eye_g).reshape(G * HIN, G * HOUT)
    wv = jnp.einsum('goi,gh->giho', W_value, eye_g).reshape(G * HIN, G * HOUT)
    return pl.pallas_call(
        _mlp_body,
        out_shape=[
            jax.ShapeDtypeStruct((B, WIDTH), jnp.float32),
            jax.ShapeDtypeStruct((B, WIDTH), jnp.float32),
        ],
    )(part, part_tc, x_res, W_pre.T, b_pre.reshape(1, WIDTH), proj, wg, wv,
      W_post.T, b_post.reshape(1, WIDTH), scale.reshape(1, WIDTH))


def kernel(x, x_res, batch, batch_size, W_pre, b_pre, W_gate, W_value, W_post,
           b_post, scale):
    del batch_size
    batch = batch.astype(jnp.int32)
    part = _sc_scatter(x, batch)
    part_tc = _tc_scatter(x, batch)
    xrn, xx = _tc_mlp(part, part_tc, x_res, W_pre, b_pre, W_gate, W_value,
                      W_post, b_post, scale)
    out = _sc_gather(xx, batch)
    return (out, xrn)
